# baseline (speedup 1.0000x reference)
import jax
import jax.numpy as jnp
from jax import lax
from jax.experimental import pallas as pl
from jax.experimental.pallas import tpu as pltpu
from jax.experimental.pallas import tpu_sc as plsc

NUM_ITEMS = 1000000
HID = 32
BATCH = 16384

NC = 2
NS = 16
L = 16
NW = NC * NS
BPW = BATCH // NW
WAVE = 4
NSTEP = BPW // (2 * WAVE)
TAIL_START = (NUM_ITEMS // 128) * 128
LAST_TILE = TAIL_START - 128


def _sc_body(idx_hbm, tp4_hbm, tc4_hbm, w_hbm, b_hbm, tailp_hbm, tailc_hbm,
             rat_out, p_out, c_out,
             idx_v, pbuf, cbuf, pf_v, cf_v, rat_v, w_v, b_v,
             tailp_v, tailc_v, semA, semB):
    c = lax.axis_index("c")
    s = lax.axis_index("s")
    wid = s * NC + c
    base = wid * BPW

    pltpu.sync_copy(idx_hbm.at[pl.ds((wid // 2) * 1024, 1024)],
                    idx_v.at[pl.ds(0, 1024)])
    loc0 = (wid % 2) * BPW
    pltpu.sync_copy(w_hbm, w_v)
    pltpu.sync_copy(b_hbm, b_v)
    pltpu.sync_copy(tailp_hbm, tailp_v)
    pltpu.sync_copy(tailc_hbm, tailc_v)

    lane = lax.iota(jnp.int32, L)
    w0 = w_v[0]
    w1 = w_v[1]
    bb = b_v[...]
    trv0 = lane // 8
    trv1 = trv0 + 2
    rv = lane % 8
    tl_lo = lane
    tl_hi = lane + L

    def issue(iv, slot, sems):
        tcb = pl.multiple_of(
            jnp.minimum(iv >> 7, LAST_TILE // 128) * 128, 128)
        pltpu.async_copy(tp4_hbm.at[:, :, pl.ds(tcb, 128)],
                         pbuf.at[slot], sems)
        pltpu.async_copy(tc4_hbm.at[:, :, pl.ds(tcb, 128)],
                         cbuf.at[slot], sems)

    def drain_wave(slots, sems):
        for slot in slots:
            pltpu.make_async_copy(tp4_hbm.at[:, :, pl.ds(0, 128)],
                                  pbuf.at[slot], sems).wait()
            pltpu.make_async_copy(tc4_hbm.at[:, :, pl.ds(0, 128)],
                                  cbuf.at[slot], sems).wait()

    def extract(iv, slot, off):
        is_tail = iv >= TAIL_START
        ccs = jnp.where(is_tail, 0, iv & 127)
        ccv = jnp.full((L,), ccs, jnp.int32)
        ev = jnp.full((L,), slot, jnp.int32)
        v0p = plsc.load_gather(pbuf, [ev, trv0, rv, ccv])
        v1p = plsc.load_gather(pbuf, [ev, trv1, rv, ccv])
        v0c = plsc.load_gather(cbuf, [ev, trv0, rv, ccv])
        v1c = plsc.load_gather(cbuf, [ev, trv1, rv, ccv])
        tloc = jnp.maximum(iv - TAIL_START, 0)
        tlv = jnp.full((L,), tloc, jnp.int32)
        tsel = jnp.full((L,), is_tail)
        v0p = jnp.where(tsel, plsc.load_gather(tailp_v, [tlv, tl_lo]), v0p)
        v1p = jnp.where(tsel, plsc.load_gather(tailp_v, [tlv, tl_hi]), v1p)
        v0c = jnp.where(tsel, plsc.load_gather(tailc_v, [tlv, tl_lo]), v0c)
        v1c = jnp.where(tsel, plsc.load_gather(tailc_v, [tlv, tl_hi]), v1c)
        pf_v[pl.ds(off, L)] = v0p
        pf_v[pl.ds(off + L, L)] = v1p
        cf_v[pl.ds(off, L)] = v0c
        cf_v[pl.ds(off + L, L)] = v1c
        t = (v0p + v0c) * w0 + (v1p + v1c) * w1
        return jnp.sum(t)

    ivec0 = idx_v[pl.ds(loc0, L)]
    for j in range(WAVE):
        issue(ivec0[j], j, semA)
    for j in range(WAVE):
        issue(ivec0[WAVE + j], WAVE + j, semB)

    def step(t, racc):
        ivx = idx_v[pl.ds(loc0 + t * 2 * WAVE, L)]
        par = t & 1
        do_issue = t < NSTEP - 1

        drain_wave(range(WAVE), semA)
        for j in range(WAVE):
            sc_val = extract(ivx[j], j, t * 2 * WAVE * HID + j * HID)
            racc = jnp.where(lane == par * 8 + j, sc_val, racc)

        @pl.when(do_issue)
        def _():
            for j in range(WAVE):
                issue(ivx[8 + j], j, semA)

        drain_wave(range(WAVE, 2 * WAVE), semB)
        for j in range(WAVE):
            sc_val = extract(ivx[WAVE + j], WAVE + j,
                             t * 2 * WAVE * HID + (WAVE + j) * HID)
            racc = jnp.where(lane == par * 8 + WAVE + j, sc_val, racc)

        @pl.when(do_issue)
        def _():
            for j in range(WAVE):
                issue(ivx[12 + j], WAVE + j, semB)

        @pl.when(par == 1)
        def _():
            rat_v[pl.ds((t - 1) * 8, L)] = (
                1.0 / (1.0 + jnp.exp(-(racc + bb))))
        return jnp.where(par == 1, jnp.zeros((L,), jnp.float32), racc)

    lax.fori_loop(0, NSTEP, step, jnp.zeros((L,), jnp.float32))

    pltpu.sync_copy(pf_v, p_out.at[pl.ds(base * HID, BPW * HID)])
    pltpu.sync_copy(cf_v, c_out.at[pl.ds(base * HID, BPW * HID)])
    pltpu.sync_copy(rat_v, rat_out.at[pl.ds(base, BPW)])


def kernel(item_indices, item_personality_table, item_commonality_table,
           user_W, user_b):
    idx1 = item_indices.astype(jnp.int32)
    tp4 = item_personality_table.T.reshape(4, 8, NUM_ITEMS)
    tc4 = item_commonality_table.T.reshape(4, 8, NUM_ITEMS)
    w2 = user_W.reshape(2, L)
    b16 = jnp.broadcast_to(user_b.reshape(1), (L,))
    tailp = item_personality_table[TAIL_START:]
    tailc = item_commonality_table[TAIL_START:]

    mesh = plsc.VectorSubcoreMesh(core_axis_name="c", subcore_axis_name="s")
    rat, p, cc = pl.kernel(
        _sc_body,
        out_type=[
            jax.ShapeDtypeStruct((BATCH,), jnp.float32),
            jax.ShapeDtypeStruct((BATCH * HID,), jnp.float32),
            jax.ShapeDtypeStruct((BATCH * HID,), jnp.float32),
        ],
        mesh=mesh,
        compiler_params=pltpu.CompilerParams(
            needs_layout_passes=False, use_tc_tiling_on_sc=True),
        scratch_types=[
            pltpu.VMEM((1024 + L,), jnp.int32),
            pltpu.VMEM((2 * WAVE, 4, 8, 128), jnp.float32),
            pltpu.VMEM((2 * WAVE, 4, 8, 128), jnp.float32),
            pltpu.VMEM((BPW * HID,), jnp.float32),
            pltpu.VMEM((BPW * HID,), jnp.float32),
            pltpu.VMEM((BPW,), jnp.float32),
            pltpu.VMEM((2, L), jnp.float32),
            pltpu.VMEM((L,), jnp.float32),
            pltpu.VMEM((128 - 64, HID), jnp.float32),
            pltpu.VMEM((128 - 64, HID), jnp.float32),
            pltpu.SemaphoreType.DMA,
            pltpu.SemaphoreType.DMA,
        ],
    )(idx1, tp4, tc4, w2, b16, tailp, tailc)
    return (rat.reshape(BATCH, 1),
            p.reshape(BATCH, HID), cc.reshape(BATCH, HID))


# speedup vs baseline: 1.1896x; 1.1896x over previous
import jax
import jax.numpy as jnp
from jax import lax
from jax.experimental import pallas as pl
from jax.experimental.pallas import tpu as pltpu
from jax.experimental.pallas import tpu_sc as plsc

NUM_ITEMS = 1000000
HID = 32
BATCH = 16384

NC = 2
NS = 16
L = 16
NW = NC * NS
BPW = BATCH // NW
WAVE = 4
NSTEP = BPW // (2 * WAVE)
CHUNK = 128
NCHUNK = BPW // CHUNK
TAIL_START = (NUM_ITEMS // 128) * 128
LAST_TILE = TAIL_START - 128
MAXTILE = LAST_TILE // 128


def _fetch_body(idx_hbm, tp4_hbm, tc4_hbm, tailp_hbm, tailc_hbm,
                p_out, c_out,
                idx_v, pbuf, cbuf, pf_v, cf_v, tailp_v, tailc_v,
                semA, semB):
    c = lax.axis_index("c")
    s = lax.axis_index("s")
    wid = s * NC + c
    base = wid * BPW

    pltpu.sync_copy(idx_hbm.at[pl.ds((wid // 2) * 1024, 1024)],
                    idx_v.at[pl.ds(0, 1024)])
    loc0 = (wid % 2) * BPW
    pltpu.sync_copy(tailp_hbm, tailp_v)
    pltpu.sync_copy(tailc_hbm, tailc_v)

    lane = lax.iota(jnp.int32, L)
    trv0 = lane // 8
    trv1 = trv0 + 2
    rv = lane % 8
    tl_lo = lane
    tl_hi = lane + L

    def tiles_conds(ivx, lo):
        tiles = [jnp.minimum(ivx[lo + j] >> 7, MAXTILE) for j in range(8)]
        conds = [jnp.full((), True)]
        for j in range(1, 8):
            conds.append(tiles[j] != tiles[j - 1])
        return tiles, conds

    def issue(tile, cond, slot, sems):
        @pl.when(cond)
        def _():
            tcb = pl.multiple_of(tile * 128, 128)
            pltpu.async_copy(tp4_hbm.at[:, :, pl.ds(tcb, 128)],
                             pbuf.at[slot], sems)
            pltpu.async_copy(tc4_hbm.at[:, :, pl.ds(tcb, 128)],
                             cbuf.at[slot], sems)

    def drain(cond, slot, sems):
        @pl.when(cond)
        def _():
            pltpu.make_async_copy(tp4_hbm.at[:, :, pl.ds(0, 128)],
                                  pbuf.at[slot], sems).wait()
            pltpu.make_async_copy(tc4_hbm.at[:, :, pl.ds(0, 128)],
                                  cbuf.at[slot], sems).wait()

    def extract(iv, slot, off):
        is_tail = iv >= TAIL_START
        ccs = jnp.where(is_tail, 0, iv & 127)
        ccv = jnp.full((L,), ccs, jnp.int32)
        ev = jnp.full((L,), slot, jnp.int32)
        v0p = plsc.load_gather(pbuf, [ev, trv0, rv, ccv])
        v1p = plsc.load_gather(pbuf, [ev, trv1, rv, ccv])
        v0c = plsc.load_gather(cbuf, [ev, trv0, rv, ccv])
        v1c = plsc.load_gather(cbuf, [ev, trv1, rv, ccv])
        tloc = jnp.maximum(iv - TAIL_START, 0)
        tlv = jnp.full((L,), tloc, jnp.int32)
        tsel = jnp.full((L,), is_tail)
        v0p = jnp.where(tsel, plsc.load_gather(tailp_v, [tlv, tl_lo]), v0p)
        v1p = jnp.where(tsel, plsc.load_gather(tailp_v, [tlv, tl_hi]), v1p)
        v0c = jnp.where(tsel, plsc.load_gather(tailc_v, [tlv, tl_lo]), v0c)
        v1c = jnp.where(tsel, plsc.load_gather(tailc_v, [tlv, tl_hi]), v1c)
        pf_v[pl.ds(off, L)] = v0p
        pf_v[pl.ds(off + L, L)] = v1p
        cf_v[pl.ds(off, L)] = v0c
        cf_v[pl.ds(off + L, L)] = v1c

    ivec0 = idx_v[pl.ds(loc0, L)]
    tiles0, conds0 = tiles_conds(ivec0, 0)
    for j in range(WAVE):
        issue(tiles0[j], conds0[j], j, semA)
    for j in range(WAVE):
        issue(tiles0[WAVE + j], conds0[WAVE + j], WAVE + j, semB)

    def step(t, carry):
        ivx = idx_v[pl.ds(loc0 + t * 2 * WAVE, L)]
        tiles, conds = tiles_conds(ivx, 0)
        do_issue = t < NSTEP - 1

        for j in range(WAVE):
            drain(conds[j], j, semA)
        for j in range(WAVE):
            drain(conds[WAVE + j], WAVE + j, semB)
        slot = jnp.full((), 0, jnp.int32)
        for j in range(8):
            slot = jnp.where(conds[j], j, slot)
            extract(ivx[j], slot, t * 2 * WAVE * HID + j * HID)

        ntiles, nconds = tiles_conds(ivx, 8)
        for j in range(WAVE):
            issue(ntiles[j], do_issue & nconds[j], j, semA)
        for j in range(WAVE):
            issue(ntiles[WAVE + j], do_issue & nconds[WAVE + j],
                  WAVE + j, semB)
        return carry

    lax.fori_loop(0, NSTEP, step, 0)

    pltpu.sync_copy(pf_v, p_out.at[pl.ds(base * HID, BPW * HID)])
    pltpu.sync_copy(cf_v, c_out.at[pl.ds(base * HID, BPW * HID)])


def _head_body(inv_hbm, ps_hbm, cs_hbm, w_hbm, b_hbm,
               rat_out, p_out, c_out,
               inv_v, p_v, c_v, rat_v, w_v, b_v, sem):
    c = lax.axis_index("c")
    s = lax.axis_index("s")
    wid = s * NC + c
    base = wid * BPW

    pltpu.sync_copy(inv_hbm.at[pl.ds(wid * NCHUNK, NCHUNK)], inv_v)
    pltpu.sync_copy(w_hbm, w_v)
    pltpu.sync_copy(b_hbm, b_v)

    cps = []
    for j in range(NCHUNK):
        cps.append(pltpu.async_copy(
            ps_hbm.at[inv_v.at[j]], p_v.at[pl.ds(j * CHUNK, CHUNK)], sem))
        cps.append(pltpu.async_copy(
            cs_hbm.at[inv_v.at[j]], c_v.at[pl.ds(j * CHUNK, CHUNK)], sem))
    for cp in cps:
        cp.wait()

    wp = pltpu.async_copy(p_v, p_out.at[pl.ds(base, BPW)], sem)
    wc = pltpu.async_copy(c_v, c_out.at[pl.ds(base, BPW)], sem)

    lane = lax.iota(jnp.int32, L)

    def group(g, carry):
        rows = g * L + lane
        acc = b_v[...]
        for d in range(HID):
            col = jnp.full((L,), d, jnp.int32)
            pv = plsc.load_gather(p_v, [rows, col])
            cv = plsc.load_gather(c_v, [rows, col])
            acc = acc + (pv + cv) * w_v[d]
        rat_v[pl.ds(g * L, L)] = 1.0 / (1.0 + jnp.exp(-acc))
        return carry

    lax.fori_loop(0, BPW // L, group, 0)

    pltpu.sync_copy(rat_v, rat_out.at[pl.ds(base, BPW)])
    wp.wait()
    wc.wait()


def kernel(item_indices, item_personality_table, item_commonality_table,
           user_W, user_b):
    idx1 = item_indices.astype(jnp.int32)
    pos = lax.iota(jnp.int32, BATCH)
    sidx, perm = lax.sort_key_val(idx1, pos)
    _, inv = lax.sort_key_val(perm, pos)
    inv2d = inv.reshape(BATCH // CHUNK, CHUNK)

    tp4 = item_personality_table.T.reshape(4, 8, NUM_ITEMS)
    tc4 = item_commonality_table.T.reshape(4, 8, NUM_ITEMS)
    tailp = item_personality_table[TAIL_START:]
    tailc = item_commonality_table[TAIL_START:]
    w_bcast = jnp.broadcast_to(user_W.reshape(HID, 1), (HID, L))
    b16 = jnp.broadcast_to(user_b.reshape(1), (L,))

    mesh = plsc.VectorSubcoreMesh(core_axis_name="c", subcore_axis_name="s")
    ps, cs = pl.kernel(
        _fetch_body,
        out_type=[
            jax.ShapeDtypeStruct((BATCH * HID,), jnp.float32),
            jax.ShapeDtypeStruct((BATCH * HID,), jnp.float32),
        ],
        mesh=mesh,
        compiler_params=pltpu.CompilerParams(
            needs_layout_passes=False, use_tc_tiling_on_sc=True),
        scratch_types=[
            pltpu.VMEM((1024 + L,), jnp.int32),
            pltpu.VMEM((2 * WAVE, 4, 8, 128), jnp.float32),
            pltpu.VMEM((2 * WAVE, 4, 8, 128), jnp.float32),
            pltpu.VMEM((BPW * HID,), jnp.float32),
            pltpu.VMEM((BPW * HID,), jnp.float32),
            pltpu.VMEM((128 - 64, HID), jnp.float32),
            pltpu.VMEM((128 - 64, HID), jnp.float32),
            pltpu.SemaphoreType.DMA,
            pltpu.SemaphoreType.DMA,
        ],
    )(sidx, tp4, tc4, tailp, tailc)

    rat, p, cc = pl.kernel(
        _head_body,
        out_type=[
            jax.ShapeDtypeStruct((BATCH,), jnp.float32),
            jax.ShapeDtypeStruct((BATCH, HID), jnp.float32),
            jax.ShapeDtypeStruct((BATCH, HID), jnp.float32),
        ],
        mesh=mesh,
        compiler_params=pltpu.CompilerParams(
            needs_layout_passes=False, use_tc_tiling_on_sc=False),
        scratch_types=[
            pltpu.VMEM((NCHUNK, CHUNK), jnp.int32),
            pltpu.VMEM((BPW, HID), jnp.float32),
            pltpu.VMEM((BPW, HID), jnp.float32),
            pltpu.VMEM((BPW,), jnp.float32),
            pltpu.VMEM((HID, L), jnp.float32),
            pltpu.VMEM((L,), jnp.float32),
            pltpu.SemaphoreType.DMA,
        ],
    )(inv2d, ps.reshape(BATCH, HID), cs.reshape(BATCH, HID), w_bcast, b16)
    return (rat.reshape(BATCH, 1), p, cc)
